# Initial kernel scaffold; baseline (speedup 1.0000x reference)
import jax
import jax.numpy as jnp
from jax.experimental import pallas as pl


def kernel(x, W1, b1, W2, b2, W3, b3):
    raise NotImplementedError("write your pallas kernel here")


# speedup vs baseline: 488.0598x; 488.0598x over previous
import jax
import jax.numpy as jnp
from jax import lax
from jax.experimental import pallas as pl
from jax.experimental.pallas import tpu as pltpu


def _make_body(H, W, T, P, n_tiles):
    N = H * W
    E = T + 2 * P

    def _body(xl_ref, xc_ref, xr_ref, w1, b1, w2, b2, w3, b3, o_ref):
        t = pl.program_id(1)
        h = jnp.concatenate([xl_ref[0], xc_ref[0], xr_ref[0]], axis=1)

        j = lax.broadcasted_iota(jnp.int32, (1, E), 1)
        g = (t * T - P + j + N) % N
        c = g % W
        r = g // W
        f32 = jnp.float32
        m_cge2 = (c >= 2).astype(f32)
        m_ceq0 = (c == 0).astype(f32)
        m_cltw = (c <= W - 2).astype(f32)
        m_rne1 = (r != 1).astype(f32)
        m_rlth = (r <= H - 2).astype(f32)
        ncol = 1.0 + m_cltw + (c != 1).astype(f32)
        nrow = 1.0 + m_rlth + m_rne1
        dis = lax.rsqrt(nrow * ncol)

        for wt, bc in ((w1, b1), (w2, b2), (w3, b3)):
            z = lax.dot_general(
                wt[...], h,
                dimension_numbers=(((1,), (0,)), ((), ())),
                preferred_element_type=jnp.float32,
            )
            a = z * dis
            rr = (a
                  + pltpu.roll(a, 1, 1) * m_cge2
                  + pltpu.roll(a, E - W + 1, 1) * m_ceq0
                  + pltpu.roll(a, E - 1, 1) * m_cltw)
            s = (rr
                 + pltpu.roll(rr, W, 1) * m_rne1
                 + pltpu.roll(rr, E - W, 1) * m_rlth)
            h = jnp.maximum(s * dis + bc[...], 0.0)
        o_ref[0] = h[:, P:P + T]

    return _body


def kernel(x, W1, b1, W2, b2, W3, b3):
    B, C, H, W = x.shape
    N = H * W
    out_c = W3.shape[1]

    T = 7168
    P = 1024
    n_tiles = N // T
    TP = T // P
    NP = N // P

    xt = x.reshape(B, C, N)
    args = (
        xt, xt, xt,
        W1.T, b1.reshape(-1, 1),
        W2.T, b2.reshape(-1, 1),
        W3.T, b3.reshape(-1, 1),
    )
    wspec = pl.BlockSpec((96, 96), lambda b, t: (0, 0))
    bspec = pl.BlockSpec((96, 1), lambda b, t: (0, 0))
    out = pl.pallas_call(
        _make_body(H, W, T, P, n_tiles),
        grid=(B, n_tiles),
        in_specs=[
            pl.BlockSpec((1, C, P), lambda b, t: (b, 0, (t * TP - 1) % NP)),
            pl.BlockSpec((1, C, T), lambda b, t: (b, 0, t)),
            pl.BlockSpec((1, C, P), lambda b, t: (b, 0, ((t + 1) * TP) % NP)),
            wspec, bspec,
            wspec, bspec,
            wspec, bspec,
        ],
        out_specs=pl.BlockSpec((1, out_c, T), lambda b, t: (b, 0, t)),
        out_shape=jax.ShapeDtypeStruct((B, out_c, N), jnp.float32),
    )(*args)
    return out.reshape(B, out_c, H, W)
